# baseline (speedup 1.0000x reference)
import functools

import jax
import jax.numpy as jnp
from jax import lax
from jax.experimental import pallas as pl
from jax.experimental.pallas import tpu as pltpu
from jax.experimental.pallas import tpu_sc as plsc

_COMMITMENT_COST = 0.25
_NT = 256
_NW = 32


def _prep_body(cb_ref, cbm2_ref, c2_ref, cbt_ref):
    cb = cb_ref[...]
    cbm2_ref[...] = (-2.0 * cb).astype(jnp.bfloat16)
    c2_ref[...] = jnp.sum(cb * cb, axis=1, keepdims=True)
    cbt_ref[...] = jnp.swapaxes(cb, 0, 1)


def _vq_body(z_ref, cbm2_ref, c2_ref, idx_ref, ls_ref):
    k = cbm2_ref.shape[0]
    nt = z_ref.shape[2]
    zt = z_ref[0]
    s_neg = jax.lax.dot_general(
        cbm2_ref[...], zt.astype(jnp.bfloat16),
        (((1,), (0,)), ((), ())),
        preferred_element_type=jnp.float32)
    z2 = jnp.sum(zt * zt, axis=0, keepdims=True)
    dist = (z2 + s_neg) + c2_ref[...]
    mins = jnp.min(dist, axis=0, keepdims=True)
    kiota = jax.lax.broadcasted_iota(jnp.int32, (k, nt), 0)
    idx_ref[0] = jnp.min(jnp.where(dist == mins, kiota, k),
                         axis=0, keepdims=True)

    @pl.when(pl.program_id(0) == 0)
    def _init():
        ls_ref[...] = jnp.zeros_like(ls_ref)

    ls_ref[...] += mins


def _sc_body(cbt_ref, idx_ref, zq_ref, cb_v, idx_v, out_v, sem0, sem1,
             *, kk, d, hw, nd):
    nb = zq_ref.shape[0]
    ngrp = hw // 16
    wid = lax.axis_index("s") * 2 + lax.axis_index("c")
    dbase = wid * nd
    for r in range(nd):
        pltpu.sync_copy(cbt_ref.at[dbase + r], cb_v.at[pl.ds(r * kk, kk)])
    pltpu.sync_copy(idx_ref, idx_v)
    sems = (sem0, sem1)
    handles = [None, None]
    for b in range(nb):
        buf = b % 2
        if handles[buf] is not None:
            handles[buf].wait()

        @plsc.parallel_loop(0, ngrp, unroll=8)
        def body(g):
            idxv = idx_v[pl.ds(b * hw + g * 16, 16)]
            for r in range(nd):
                v = plsc.load_gather(cb_v, [idxv + r * kk])
                out_v[buf, r, pl.ds(g * 16, 16)] = v

        handles[buf] = pltpu.async_copy(
            out_v.at[buf], zq_ref.at[b, pl.ds(dbase, nd)], sems[buf])
    handles[0].wait()
    handles[1].wait()


def kernel(z, codebook):
    b, d, h, w = z.shape
    kk = codebook.shape[0]
    hw = h * w
    nt = min(_NT, hw)
    nblk = hw // nt
    grid = b * nblk
    z3 = z.reshape(b, d, hw)

    cbm2, c2, cbt = pl.pallas_call(
        _prep_body,
        out_shape=[
            jax.ShapeDtypeStruct((kk, d), jnp.bfloat16),
            jax.ShapeDtypeStruct((kk, 1), jnp.float32),
            jax.ShapeDtypeStruct((d, kk), jnp.float32),
        ],
    )(codebook)

    idx, ls = pl.pallas_call(
        _vq_body,
        grid=(grid,),
        in_specs=[
            pl.BlockSpec((1, d, nt), lambda i: (i // nblk, 0, i % nblk)),
            pl.BlockSpec((kk, d), lambda i: (0, 0)),
            pl.BlockSpec((kk, 1), lambda i: (0, 0)),
        ],
        out_specs=[
            pl.BlockSpec((1, 1, nt), lambda i: (i, 0, 0)),
            pl.BlockSpec((1, nt), lambda i: (0, 0)),
        ],
        out_shape=[
            jax.ShapeDtypeStruct((grid, 1, nt), jnp.int32),
            jax.ShapeDtypeStruct((1, nt), jnp.float32),
        ],
    )(z3, cbm2, c2)

    nd = d // _NW
    sc = pl.kernel(
        functools.partial(_sc_body, kk=kk, d=d, hw=hw, nd=nd),
        out_type=jax.ShapeDtypeStruct((b, d, hw), jnp.float32),
        mesh=plsc.VectorSubcoreMesh(core_axis_name="c", subcore_axis_name="s"),
        compiler_params=pltpu.CompilerParams(needs_layout_passes=False),
        scratch_types=[
            pltpu.VMEM((nd * kk,), jnp.float32),
            pltpu.VMEM((b * hw,), jnp.int32),
            pltpu.VMEM((2, nd, hw), jnp.float32),
            pltpu.SemaphoreType.DMA,
            pltpu.SemaphoreType.DMA,
        ],
    )
    zq2 = sc(cbt, idx.reshape(-1))

    zq_out = zq2.reshape(b, d, h, w)
    idx_out = idx.reshape(b, h, w)
    mse = jnp.sum(ls) / (b * d * hw)
    vq_loss = mse + _COMMITMENT_COST * mse
    return zq_out, idx_out, vq_loss


# speedup vs baseline: 1.3299x; 1.3299x over previous
import functools

import jax
import jax.numpy as jnp
from jax import lax
from jax.experimental import pallas as pl
from jax.experimental.pallas import tpu as pltpu
from jax.experimental.pallas import tpu_sc as plsc

_COMMITMENT_COST = 0.25
_NT = 256
_NW = 32
_CHUNK = 128


def _prep_body(cb_ref, cbm2t_ref, c2_ref):
    cb = cb_ref[...]
    cbm2t_ref[...] = jnp.swapaxes((-2.0 * cb).astype(jnp.bfloat16), 0, 1)
    c2 = jnp.sum(cb * cb, axis=1, keepdims=True)
    c2_ref[...] = jnp.swapaxes(c2, 0, 1)


def _vq_body(z_ref, cbm2t_ref, c2_ref, idx_ref, ls_ref):
    k = cbm2t_ref.shape[1]
    nt = z_ref.shape[0]
    zt = z_ref[...]
    s_neg = jax.lax.dot_general(
        zt.astype(jnp.bfloat16), cbm2t_ref[...],
        (((1,), (0,)), ((), ())),
        preferred_element_type=jnp.float32)
    z2 = jnp.sum(zt * zt, axis=1, keepdims=True)
    dist = (z2 + s_neg) + c2_ref[...]
    mins = jnp.min(dist, axis=1, keepdims=True)
    kiota = jax.lax.broadcasted_iota(jnp.int32, (nt, k), 1)
    idx = jnp.min(jnp.where(dist == mins, kiota, k),
                  axis=1, keepdims=True)
    idx_ref[0] = jnp.swapaxes(idx, 0, 1)

    @pl.when(pl.program_id(0) == 0)
    def _init():
        ls_ref[...] = jnp.zeros_like(ls_ref)

    ls_ref[...] += mins


def _sc_body(cb_ref, idx_ref, zq_ref, idx_v, rows_v, sem0, sem1, so0, so1,
             *, n, chunk):
    per_w = n // _NW
    nchunk = per_w // chunk
    wid = lax.axis_index("s") * 2 + lax.axis_index("c")
    base = wid * per_w
    pltpu.sync_copy(idx_ref.at[pl.ds(base, per_w)], idx_v)
    gsems = (sem0, sem1)
    osems = (so0, so1)
    out_h = [None, None]
    for c in range(nchunk):
        buf = c % 2
        if out_h[buf] is not None:
            out_h[buf].wait()
        pltpu.async_copy(
            cb_ref.at[idx_v.at[pl.ds(c * chunk, chunk)]],
            rows_v.at[buf], gsems[buf]).wait()
        out_h[buf] = pltpu.async_copy(
            rows_v.at[buf], zq_ref.at[pl.ds(base + c * chunk, chunk)],
            osems[buf])
    out_h[0].wait()
    out_h[1].wait()


def kernel(z, codebook):
    b, d, h, w = z.shape
    kk = codebook.shape[0]
    hw = h * w
    n = b * hw
    nt = min(_NT, n)
    grid = n // nt
    zr = z.transpose(0, 2, 3, 1).reshape(n, d)

    cbm2t, c2 = pl.pallas_call(
        _prep_body,
        out_shape=[
            jax.ShapeDtypeStruct((d, kk), jnp.bfloat16),
            jax.ShapeDtypeStruct((1, kk), jnp.float32),
        ],
    )(codebook)

    idx, ls = pl.pallas_call(
        _vq_body,
        grid=(grid,),
        in_specs=[
            pl.BlockSpec((nt, d), lambda i: (i, 0)),
            pl.BlockSpec((d, kk), lambda i: (0, 0)),
            pl.BlockSpec((1, kk), lambda i: (0, 0)),
        ],
        out_specs=[
            pl.BlockSpec((1, 1, nt), lambda i: (i, 0, 0)),
            pl.BlockSpec((nt, 1), lambda i: (0, 0)),
        ],
        out_shape=[
            jax.ShapeDtypeStruct((grid, 1, nt), jnp.int32),
            jax.ShapeDtypeStruct((nt, 1), jnp.float32),
        ],
    )(zr, cbm2t, c2)

    sc = pl.kernel(
        functools.partial(_sc_body, n=n, chunk=_CHUNK),
        out_type=jax.ShapeDtypeStruct((n, d), jnp.float32),
        mesh=plsc.VectorSubcoreMesh(core_axis_name="c", subcore_axis_name="s"),
        compiler_params=pltpu.CompilerParams(needs_layout_passes=False),
        scratch_types=[
            pltpu.VMEM((n // _NW,), jnp.int32),
            pltpu.VMEM((2, _CHUNK, d), jnp.float32),
            pltpu.SemaphoreType.DMA,
            pltpu.SemaphoreType.DMA,
            pltpu.SemaphoreType.DMA,
            pltpu.SemaphoreType.DMA,
        ],
    )
    zq = sc(codebook, idx.reshape(-1))

    zq_out = zq.reshape(b, h, w, d).transpose(0, 3, 1, 2)
    idx_out = idx.reshape(b, h, w)
    mse = jnp.sum(ls) / (b * d * hw)
    vq_loss = mse + _COMMITMENT_COST * mse
    return zq_out, idx_out, vq_loss


# speedup vs baseline: 1.4474x; 1.0884x over previous
import functools

import jax
import jax.numpy as jnp
from jax import lax
from jax.experimental import pallas as pl
from jax.experimental.pallas import tpu as pltpu
from jax.experimental.pallas import tpu_sc as plsc

_COMMITMENT_COST = 0.25
_NT = 512
_NW = 32
_CHUNK = 128


def _prep_body(cb_ref, cbm2t_ref, c2_ref):
    cb = cb_ref[...]
    cbm2t_ref[...] = jnp.swapaxes((-2.0 * cb).astype(jnp.bfloat16), 0, 1)
    c2 = jnp.sum(cb * cb, axis=1, keepdims=True)
    c2_ref[...] = jnp.swapaxes(c2, 0, 1)


def _vq_body(z_ref, cbm2t_ref, c2_ref, idx_ref, ls_ref):
    k = cbm2t_ref.shape[1]
    nt = z_ref.shape[0]
    zt = z_ref[...]
    s_neg = jax.lax.dot_general(
        zt.astype(jnp.bfloat16), cbm2t_ref[...],
        (((1,), (0,)), ((), ())),
        preferred_element_type=jnp.float32)
    z2 = jnp.sum(zt * zt, axis=1, keepdims=True)
    dist = (z2 + s_neg) + c2_ref[...]
    mins = jnp.min(dist, axis=1, keepdims=True)
    kiota = jax.lax.broadcasted_iota(jnp.int32, (nt, k), 1)
    idx = jnp.min(jnp.where(dist == mins, kiota, k),
                  axis=1, keepdims=True)
    idx_ref[0] = jnp.swapaxes(idx, 0, 1)

    @pl.when(pl.program_id(0) == 0)
    def _init():
        ls_ref[...] = jnp.zeros_like(ls_ref)

    ls_ref[...] += mins


def _sc_body(cb_ref, idx_ref, zq_ref, idx_v, rows_v, sem0, sem1, so0, so1,
             *, n, chunk):
    per_w = n // _NW
    nchunk = per_w // chunk
    wid = lax.axis_index("s") * 2 + lax.axis_index("c")
    base = wid * per_w
    pltpu.sync_copy(idx_ref.at[pl.ds(base, per_w)], idx_v)
    gsems = (sem0, sem1)
    osems = (so0, so1)
    out_h = [None, None]
    for c in range(nchunk):
        buf = c % 2
        if out_h[buf] is not None:
            out_h[buf].wait()
        pltpu.async_copy(
            cb_ref.at[idx_v.at[pl.ds(c * chunk, chunk)]],
            rows_v.at[buf], gsems[buf]).wait()
        out_h[buf] = pltpu.async_copy(
            rows_v.at[buf], zq_ref.at[pl.ds(base + c * chunk, chunk)],
            osems[buf])
    out_h[0].wait()
    out_h[1].wait()


def kernel(z, codebook):
    b, d, h, w = z.shape
    kk = codebook.shape[0]
    hw = h * w
    n = b * hw
    nt = min(_NT, n)
    grid = n // nt
    zr = z.transpose(0, 2, 3, 1).reshape(n, d)

    cbm2t, c2 = pl.pallas_call(
        _prep_body,
        out_shape=[
            jax.ShapeDtypeStruct((d, kk), jnp.bfloat16),
            jax.ShapeDtypeStruct((1, kk), jnp.float32),
        ],
    )(codebook)

    idx, ls = pl.pallas_call(
        _vq_body,
        grid=(grid,),
        in_specs=[
            pl.BlockSpec((nt, d), lambda i: (i, 0)),
            pl.BlockSpec((d, kk), lambda i: (0, 0)),
            pl.BlockSpec((1, kk), lambda i: (0, 0)),
        ],
        out_specs=[
            pl.BlockSpec((1, 1, nt), lambda i: (i, 0, 0)),
            pl.BlockSpec((nt, 1), lambda i: (0, 0)),
        ],
        out_shape=[
            jax.ShapeDtypeStruct((grid, 1, nt), jnp.int32),
            jax.ShapeDtypeStruct((nt, 1), jnp.float32),
        ],
    )(zr, cbm2t, c2)

    sc = pl.kernel(
        functools.partial(_sc_body, n=n, chunk=_CHUNK),
        out_type=jax.ShapeDtypeStruct((n, d), jnp.float32),
        mesh=plsc.VectorSubcoreMesh(core_axis_name="c", subcore_axis_name="s"),
        compiler_params=pltpu.CompilerParams(needs_layout_passes=False),
        scratch_types=[
            pltpu.VMEM((n // _NW,), jnp.int32),
            pltpu.VMEM((2, _CHUNK, d), jnp.float32),
            pltpu.SemaphoreType.DMA,
            pltpu.SemaphoreType.DMA,
            pltpu.SemaphoreType.DMA,
            pltpu.SemaphoreType.DMA,
        ],
    )
    zq = sc(codebook, idx.reshape(-1))

    zq_out = zq.reshape(b, h, w, d).transpose(0, 3, 1, 2)
    idx_out = idx.reshape(b, h, w)
    mse = jnp.sum(ls) / (b * d * hw)
    vq_loss = mse + _COMMITMENT_COST * mse
    return zq_out, idx_out, vq_loss


# speedup vs baseline: 1.5098x; 1.0431x over previous
import functools

import jax
import jax.numpy as jnp
from jax import lax
from jax.experimental import pallas as pl
from jax.experimental.pallas import tpu as pltpu
from jax.experimental.pallas import tpu_sc as plsc

_COMMITMENT_COST = 0.25
_NT = 1024
_NW = 32
_CHUNK = 128


def _prep_body(cb_ref, cbm2t_ref, c2_ref):
    cb = cb_ref[...]
    cbm2t_ref[...] = jnp.swapaxes((-2.0 * cb).astype(jnp.bfloat16), 0, 1)
    c2 = jnp.sum(cb * cb, axis=1, keepdims=True)
    c2_ref[...] = jnp.swapaxes(c2, 0, 1)


def _vq_body(z_ref, cbm2t_ref, c2_ref, idx_ref, ls_ref):
    k = cbm2t_ref.shape[1]
    nt = z_ref.shape[0]
    zt = z_ref[...]
    s_neg = jax.lax.dot_general(
        zt.astype(jnp.bfloat16), cbm2t_ref[...],
        (((1,), (0,)), ((), ())),
        preferred_element_type=jnp.float32)
    z2 = jnp.sum(zt * zt, axis=1, keepdims=True)
    dist = (z2 + s_neg) + c2_ref[...]
    mins = jnp.min(dist, axis=1, keepdims=True)
    kiota = jax.lax.broadcasted_iota(jnp.int32, (nt, k), 1)
    idx = jnp.min(jnp.where(dist == mins, kiota, k),
                  axis=1, keepdims=True)
    idx_ref[0] = jnp.swapaxes(idx, 0, 1)

    @pl.when(pl.program_id(0) == 0)
    def _init():
        ls_ref[...] = jnp.zeros_like(ls_ref)

    ls_ref[...] += mins


def _sc_body(cb_ref, idx_ref, zq_ref, idx_v, rows_v, sem0, sem1, so0, so1,
             *, n, chunk):
    per_w = n // _NW
    nchunk = per_w // chunk
    wid = lax.axis_index("s") * 2 + lax.axis_index("c")
    base = wid * per_w
    pltpu.sync_copy(idx_ref.at[pl.ds(base, per_w)], idx_v)
    gsems = (sem0, sem1)
    osems = (so0, so1)
    out_h = [None, None]
    for c in range(nchunk):
        buf = c % 2
        if out_h[buf] is not None:
            out_h[buf].wait()
        pltpu.async_copy(
            cb_ref.at[idx_v.at[pl.ds(c * chunk, chunk)]],
            rows_v.at[buf], gsems[buf]).wait()
        out_h[buf] = pltpu.async_copy(
            rows_v.at[buf], zq_ref.at[pl.ds(base + c * chunk, chunk)],
            osems[buf])
    out_h[0].wait()
    out_h[1].wait()


def kernel(z, codebook):
    b, d, h, w = z.shape
    kk = codebook.shape[0]
    hw = h * w
    n = b * hw
    nt = min(_NT, n)
    grid = n // nt
    zr = z.transpose(0, 2, 3, 1).reshape(n, d)

    cbm2t, c2 = pl.pallas_call(
        _prep_body,
        out_shape=[
            jax.ShapeDtypeStruct((d, kk), jnp.bfloat16),
            jax.ShapeDtypeStruct((1, kk), jnp.float32),
        ],
    )(codebook)

    idx, ls = pl.pallas_call(
        _vq_body,
        grid=(grid,),
        in_specs=[
            pl.BlockSpec((nt, d), lambda i: (i, 0)),
            pl.BlockSpec((d, kk), lambda i: (0, 0)),
            pl.BlockSpec((1, kk), lambda i: (0, 0)),
        ],
        out_specs=[
            pl.BlockSpec((1, 1, nt), lambda i: (i, 0, 0)),
            pl.BlockSpec((nt, 1), lambda i: (0, 0)),
        ],
        out_shape=[
            jax.ShapeDtypeStruct((grid, 1, nt), jnp.int32),
            jax.ShapeDtypeStruct((nt, 1), jnp.float32),
        ],
    )(zr, cbm2t, c2)

    sc = pl.kernel(
        functools.partial(_sc_body, n=n, chunk=_CHUNK),
        out_type=jax.ShapeDtypeStruct((n, d), jnp.float32),
        mesh=plsc.VectorSubcoreMesh(core_axis_name="c", subcore_axis_name="s"),
        compiler_params=pltpu.CompilerParams(needs_layout_passes=False),
        scratch_types=[
            pltpu.VMEM((n // _NW,), jnp.int32),
            pltpu.VMEM((2, _CHUNK, d), jnp.float32),
            pltpu.SemaphoreType.DMA,
            pltpu.SemaphoreType.DMA,
            pltpu.SemaphoreType.DMA,
            pltpu.SemaphoreType.DMA,
        ],
    )
    zq = sc(codebook, idx.reshape(-1))

    zq_out = zq.reshape(b, h, w, d).transpose(0, 3, 1, 2)
    idx_out = idx.reshape(b, h, w)
    mse = jnp.sum(ls) / (b * d * hw)
    vq_loss = mse + _COMMITMENT_COST * mse
    return zq_out, idx_out, vq_loss
